# Initial kernel scaffold; baseline (speedup 1.0000x reference)
import jax
import jax.numpy as jnp
from jax.experimental import pallas as pl


def kernel(features, edge_index, W1, b1, W2, b2, W3, b3, Wf, bf):
    raise NotImplementedError("write your pallas kernel here")


# speedup vs baseline: 6.3496x; 6.3496x over previous
import jax
import jax.numpy as jnp
from jax import lax
from jax.experimental import pallas as pl
from jax.experimental.pallas import tpu as pltpu
from jax.experimental.pallas import tpu_sc as plsc

_N = 10000
_E = 640000
_DIN = 602
_H = 128
_C = 41

_NC = 2
_NS = 16
_N2 = 10240
_DSTRIPE = _N2 // _NS
_EC = _E // (_NC * _NS)
_CH = 80
_NCHUNK = _EC // _CH
_NR = 10240
_RSTRIPE = _NR // _NS
_R = 1000
_G = _N // _R

_mesh = plsc.VectorSubcoreMesh(core_axis_name="c", subcore_axis_name="s")


def _deg_body(src_hbm, dst_hbm, out_hbm, sidx, didx, ones_v, zbuf, acc_o, acc_i):
    cid = lax.axis_index("c")
    sid = lax.axis_index("s")
    base = cid * (_E // _NC) + sid * _EC

    for j in range(_CH // 16):
        ones_v[pl.ds(j * 16, 16)] = jnp.ones((16,), jnp.float32)
    for j in range(_DSTRIPE // 16):
        zbuf[pl.ds(j * 16, 16)] = jnp.zeros((16,), jnp.float32)
    d0 = sid * _DSTRIPE
    pltpu.sync_copy(zbuf, acc_o.at[pl.ds(d0, _DSTRIPE)])
    pltpu.sync_copy(zbuf, acc_i.at[pl.ds(d0, _DSTRIPE)])
    plsc.subcore_barrier()

    def chunk(i, carry):
        off = base + i * _CH
        pltpu.sync_copy(src_hbm.at[pl.ds(off, _CH)], sidx)
        pltpu.sync_copy(dst_hbm.at[pl.ds(off, _CH)], didx)
        pltpu.sync_copy(ones_v, acc_o.at[sidx], add=True)
        pltpu.sync_copy(ones_v, acc_i.at[didx], add=True)
        return carry

    lax.fori_loop(0, _NCHUNK, chunk, 0)
    plsc.subcore_barrier()

    pltpu.sync_copy(acc_o.at[pl.ds(d0, _DSTRIPE)],
                    out_hbm.at[cid, 0, pl.ds(d0, _DSTRIPE)])
    pltpu.sync_copy(acc_i.at[pl.ds(d0, _DSTRIPE)],
                    out_hbm.at[cid, 1, pl.ds(d0, _DSTRIPE)])


_deg = pl.kernel(
    _deg_body,
    out_type=jax.ShapeDtypeStruct((_NC, 2, _N2), jnp.float32),
    mesh=_mesh,
    scratch_types=[
        pltpu.VMEM((_CH,), jnp.int32),
        pltpu.VMEM((_CH,), jnp.int32),
        pltpu.VMEM((_CH,), jnp.float32),
        pltpu.VMEM((_DSTRIPE,), jnp.float32),
        pltpu.VMEM_SHARED((_N2,), jnp.float32),
        pltpu.VMEM_SHARED((_N2,), jnp.float32),
    ],
)


def _agg_body(g_hbm, src_hbm, dst_hbm, zeros_hbm, out_hbm,
              sidx, didx, rows, acc, sem):
    cid = lax.axis_index("c")
    sid = lax.axis_index("s")
    base = cid * (_E // _NC) + sid * _EC

    r0 = sid * _RSTRIPE
    pltpu.sync_copy(zeros_hbm.at[pl.ds(r0, _RSTRIPE)],
                    acc.at[pl.ds(r0, _RSTRIPE)])
    plsc.subcore_barrier()

    def chunk(i, carry):
        off = base + i * _CH
        pltpu.sync_copy(src_hbm.at[pl.ds(off, _CH)], sidx)
        pltpu.sync_copy(dst_hbm.at[pl.ds(off, _CH)], didx)
        pltpu.async_copy(g_hbm.at[sidx], rows, sem).wait()
        pltpu.sync_copy(rows, acc.at[didx], add=True)
        return carry

    lax.fori_loop(0, _NCHUNK, chunk, 0)
    plsc.subcore_barrier()

    pltpu.sync_copy(acc.at[pl.ds(r0, _RSTRIPE)],
                    out_hbm.at[cid, pl.ds(r0, _RSTRIPE)])


_agg = pl.kernel(
    _agg_body,
    out_type=jax.ShapeDtypeStruct((_NC, _NR, _H), jnp.float32),
    mesh=_mesh,
    scratch_types=[
        pltpu.VMEM((_CH,), jnp.int32),
        pltpu.VMEM((_CH,), jnp.int32),
        pltpu.VMEM((_CH, _H), jnp.float32),
        pltpu.VMEM_SHARED((_NR, _H), jnp.float32),
        pltpu.SemaphoreType.DMA,
    ],
)


def _tca_body(feat_ref, w1_ref, degp_ref, g_ref, ns_ref, nd_ref):
    dp = degp_ref[...]
    deg_o = dp[:, 0:1] + dp[:, 2:3]
    deg_i = dp[:, 1:2] + dp[:, 3:4]
    ns = lax.rsqrt(jnp.maximum(deg_o, 1.0))
    nd = lax.rsqrt(jnp.maximum(deg_i, 1.0))
    ns_ref[...] = ns
    nd_ref[...] = nd
    g_ref[...] = jnp.dot(feat_ref[...], w1_ref[...],
                         preferred_element_type=jnp.float32) * ns


_tca = pl.pallas_call(
    _tca_body,
    grid=(_G,),
    in_specs=[
        pl.BlockSpec((_R, _DIN), lambda i: (i, 0)),
        pl.BlockSpec((_DIN, _H), lambda i: (0, 0)),
        pl.BlockSpec((_R, 4), lambda i: (i, 0)),
    ],
    out_specs=[
        pl.BlockSpec((_R, _H), lambda i: (i, 0)),
        pl.BlockSpec((_R, 1), lambda i: (i, 0)),
        pl.BlockSpec((_R, 1), lambda i: (i, 0)),
    ],
    out_shape=[
        jax.ShapeDtypeStruct((_N, _H), jnp.float32),
        jax.ShapeDtypeStruct((_N, 1), jnp.float32),
        jax.ShapeDtypeStruct((_N, 1), jnp.float32),
    ],
)


def _tcb_body_g(p_ref, nd_ref, ns_ref, b_ref, w_ref, g_ref):
    p = p_ref[...]
    h = jnp.maximum((p[0] + p[1]) * nd_ref[...] + b_ref[...], 0.0)
    g_ref[...] = jnp.dot(h, w_ref[...],
                         preferred_element_type=jnp.float32) * ns_ref[...]


def _tcb_body_hg(p_ref, nd_ref, ns_ref, b_ref, w_ref, h_ref, g_ref):
    p = p_ref[...]
    h = jnp.maximum((p[0] + p[1]) * nd_ref[...] + b_ref[...], 0.0)
    h_ref[...] = h
    g_ref[...] = jnp.dot(h, w_ref[...],
                         preferred_element_type=jnp.float32) * ns_ref[...]


_tcb_in_specs = [
    pl.BlockSpec((_NC, _R, _H), lambda i: (0, i, 0)),
    pl.BlockSpec((_R, 1), lambda i: (i, 0)),
    pl.BlockSpec((_R, 1), lambda i: (i, 0)),
    pl.BlockSpec((1, _H), lambda i: (0, 0)),
    pl.BlockSpec((_H, _H), lambda i: (0, 0)),
]

_tcb1 = pl.pallas_call(
    _tcb_body_g,
    grid=(_G,),
    in_specs=_tcb_in_specs,
    out_specs=pl.BlockSpec((_R, _H), lambda i: (i, 0)),
    out_shape=jax.ShapeDtypeStruct((_N, _H), jnp.float32),
)

_tcb2 = pl.pallas_call(
    _tcb_body_hg,
    grid=(_G,),
    in_specs=_tcb_in_specs,
    out_specs=[
        pl.BlockSpec((_R, _H), lambda i: (i, 0)),
        pl.BlockSpec((_R, _H), lambda i: (i, 0)),
    ],
    out_shape=[
        jax.ShapeDtypeStruct((_N, _H), jnp.float32),
        jax.ShapeDtypeStruct((_N, _H), jnp.float32),
    ],
)


def _tcc_body(p_ref, nd_ref, b_ref, h2_ref, wf_ref, bf_ref, o_ref):
    p = p_ref[...]
    h3 = jnp.maximum((p[0] + p[1]) * nd_ref[...] + b_ref[...], 0.0)
    y = jnp.maximum(h3 + h2_ref[...], 0.0)
    o_ref[...] = jnp.dot(y, wf_ref[...],
                         preferred_element_type=jnp.float32) + bf_ref[...]


_tcc = pl.pallas_call(
    _tcc_body,
    grid=(_G,),
    in_specs=[
        pl.BlockSpec((_NC, _R, _H), lambda i: (0, i, 0)),
        pl.BlockSpec((_R, 1), lambda i: (i, 0)),
        pl.BlockSpec((1, _H), lambda i: (0, 0)),
        pl.BlockSpec((_R, _H), lambda i: (i, 0)),
        pl.BlockSpec((_H, _C), lambda i: (0, 0)),
        pl.BlockSpec((1, _C), lambda i: (0, 0)),
    ],
    out_specs=pl.BlockSpec((_R, _C), lambda i: (i, 0)),
    out_shape=jax.ShapeDtypeStruct((_N, _C), jnp.float32),
)


def kernel(features, edge_index, W1, b1, W2, b2, W3, b3, Wf, bf):
    src = edge_index[0]
    dst = edge_index[1]

    degp = _deg(src, dst)
    degp_t = jnp.transpose(degp.reshape(_NC * 2, _N2))

    g1, ns, nd = _tca(features, W1, degp_t)

    zeros = jnp.zeros((_NR, _H), jnp.float32)
    p1 = _agg(g1, src, dst, zeros)
    g2 = _tcb1(p1, nd, ns, b1.reshape(1, _H), W2)
    p2 = _agg(g2, src, dst, zeros)
    h2, g3 = _tcb2(p2, nd, ns, b2.reshape(1, _H), W3)
    p3 = _agg(g3, src, dst, zeros)
    out = _tcc(p3, nd, b3.reshape(1, _H), h2, Wf, bf.reshape(1, _C))
    return out


# speedup vs baseline: 15.9293x; 2.5087x over previous
import jax
import jax.numpy as jnp
from jax import lax
from jax.experimental import pallas as pl
from jax.experimental.pallas import tpu as pltpu
from jax.experimental.pallas import tpu_sc as plsc

_N = 10000
_E = 640000
_DIN = 602
_H = 128
_C = 41

_NC = 2
_NS = 16
_NW = _NC * _NS
_N2 = 10240
_DSTRIPE = _N2 // _NS
_EC = _E // _NW
_CH = 125
_NCHUNK = _EC // _CH
_IB = 16
_NBLK = _NCHUNK // _IB
_NR = 10240
_RSTRIPE = _NR // _NS
_ZR = 64
_DLAG = 8
_R = 1000
_G = _N // _R

_mesh = plsc.VectorSubcoreMesh(core_axis_name="c", subcore_axis_name="s")


def _deg_body(src3_hbm, dst3_hbm, out_hbm, sidx2, didx2, ones_v, zbuf,
              acc_o, acc_i, sem_o, sem_i):
    cid = lax.axis_index("c")
    sid = lax.axis_index("s")
    wid = cid * _NS + sid

    pltpu.sync_copy(src3_hbm.at[wid], sidx2)
    pltpu.sync_copy(dst3_hbm.at[wid], didx2)

    for j in range(128 // 16):
        ones_v[pl.ds(j * 16, 16)] = jnp.ones((16,), jnp.float32)
    for j in range(_DSTRIPE // 16):
        zbuf[pl.ds(j * 16, 16)] = jnp.zeros((16,), jnp.float32)
    d0 = sid * _DSTRIPE
    pltpu.sync_copy(zbuf, acc_o.at[pl.ds(d0, _DSTRIPE)])
    pltpu.sync_copy(zbuf, acc_i.at[pl.ds(d0, _DSTRIPE)])
    plsc.subcore_barrier()

    ones_ch = ones_v.at[pl.ds(0, _CH)]

    def chunk(j, carry):
        pltpu.async_copy(ones_ch, acc_o.at[sidx2.at[j]], sem_o, add=True)
        pltpu.async_copy(ones_ch, acc_i.at[didx2.at[j]], sem_i, add=True)

        @pl.when(j >= _DLAG)
        def _():
            pltpu.make_async_copy(ones_ch, acc_o.at[sidx2.at[j]], sem_o).wait()
            pltpu.make_async_copy(ones_ch, acc_i.at[didx2.at[j]], sem_i).wait()

        return carry

    lax.fori_loop(0, _NCHUNK, chunk, 0)
    for _ in range(_DLAG):
        pltpu.make_async_copy(ones_ch, acc_o.at[sidx2.at[0]], sem_o).wait()
        pltpu.make_async_copy(ones_ch, acc_i.at[didx2.at[0]], sem_i).wait()
    plsc.subcore_barrier()

    pltpu.sync_copy(acc_o.at[pl.ds(d0, _DSTRIPE)],
                    out_hbm.at[0, cid, pl.ds(d0, _DSTRIPE)])
    pltpu.sync_copy(acc_i.at[pl.ds(d0, _DSTRIPE)],
                    out_hbm.at[1, cid, pl.ds(d0, _DSTRIPE)])


_deg = pl.kernel(
    _deg_body,
    out_type=jax.ShapeDtypeStruct((2, _NC, _N2), jnp.float32),
    mesh=_mesh,
    scratch_types=[
        pltpu.VMEM((_NCHUNK, _CH), jnp.int32),
        pltpu.VMEM((_NCHUNK, _CH), jnp.int32),
        pltpu.VMEM((128,), jnp.float32),
        pltpu.VMEM((_DSTRIPE,), jnp.float32),
        pltpu.VMEM_SHARED((_N2,), jnp.float32),
        pltpu.VMEM_SHARED((_N2,), jnp.float32),
        pltpu.SemaphoreType.DMA,
        pltpu.SemaphoreType.DMA,
    ],
)


def _agg_body(g_hbm, src3_hbm, dst3_hbm, out_hbm,
              s0, s1, d0i, d1i, b0, b1, zrows, acc, sem_i, sem_g, sem_s):
    cid = lax.axis_index("c")
    sid = lax.axis_index("s")
    wid = cid * _NS + sid
    sblk = [s0, s1]
    dblk = [d0i, d1i]
    bufs = [b0, b1]

    def zrow(i, carry):
        for j in range(_H // 16):
            zrows[i, pl.ds(j * 16, 16)] = jnp.zeros((16,), jnp.float32)
        return carry

    lax.fori_loop(0, _ZR, zrow, 0)
    r0 = sid * _RSTRIPE
    for k in range(_RSTRIPE // _ZR):
        pltpu.async_copy(zrows, acc.at[pl.ds(r0 + k * _ZR, _ZR)], sem_g)
    pltpu.async_copy(src3_hbm.at[wid, pl.ds(0, _IB)], s0, sem_i)
    pltpu.async_copy(dst3_hbm.at[wid, pl.ds(0, _IB)], d0i, sem_i)
    for k in range(_RSTRIPE // _ZR):
        pltpu.make_async_copy(zrows, acc.at[pl.ds(r0 + k * _ZR, _ZR)],
                              sem_g).wait()
    plsc.subcore_barrier()

    def block(kb, p):
        sb = sblk[p]
        db = dblk[p]
        j0 = kb * _IB
        pltpu.make_async_copy(src3_hbm.at[wid, pl.ds(j0, _IB)], sb,
                              sem_i).wait()
        pltpu.make_async_copy(dst3_hbm.at[wid, pl.ds(j0, _IB)], db,
                              sem_i).wait()

        @pl.when(kb + 1 < _NBLK)
        def _():
            pltpu.async_copy(src3_hbm.at[wid, pl.ds(j0 + _IB, _IB)],
                             sblk[1 - p], sem_i)
            pltpu.async_copy(dst3_hbm.at[wid, pl.ds(j0 + _IB, _IB)],
                             dblk[1 - p], sem_i)

        pltpu.async_copy(g_hbm.at[sb.at[0]], bufs[0], sem_g)

        for jj in range(_IB):
            bj = jj % 2
            j = j0 + jj
            pltpu.make_async_copy(g_hbm.at[sb.at[jj]], bufs[bj],
                                  sem_g).wait()
            pltpu.async_copy(bufs[bj], acc.at[db.at[jj]], sem_s, add=True)

            @pl.when(j >= 1)
            def _():
                pltpu.make_async_copy(bufs[1 - bj], acc.at[db.at[jj]],
                                      sem_s).wait()

            if jj + 1 < _IB:
                pltpu.async_copy(g_hbm.at[sb.at[jj + 1]], bufs[1 - bj],
                                 sem_g)

    def outer(jo, carry):
        block(jo * 2, 0)
        block(jo * 2 + 1, 1)
        return carry

    lax.fori_loop(0, _NBLK // 2, outer, 0)
    pltpu.make_async_copy(b0, acc.at[d0i.at[0]], sem_s).wait()
    plsc.subcore_barrier()

    pltpu.sync_copy(acc.at[pl.ds(r0, _RSTRIPE)],
                    out_hbm.at[cid, pl.ds(r0, _RSTRIPE)])


_agg = pl.kernel(
    _agg_body,
    out_type=jax.ShapeDtypeStruct((_NC, _NR, _H), jnp.float32),
    mesh=_mesh,
    scratch_types=[
        pltpu.VMEM((_IB, _CH), jnp.int32),
        pltpu.VMEM((_IB, _CH), jnp.int32),
        pltpu.VMEM((_IB, _CH), jnp.int32),
        pltpu.VMEM((_IB, _CH), jnp.int32),
        pltpu.VMEM((_CH, _H), jnp.float32),
        pltpu.VMEM((_CH, _H), jnp.float32),
        pltpu.VMEM((_ZR, _H), jnp.float32),
        pltpu.VMEM_SHARED((_NR, _H), jnp.float32),
        pltpu.SemaphoreType.DMA,
        pltpu.SemaphoreType.DMA,
        pltpu.SemaphoreType.DMA,
    ],
)


def _tca_body(feat_ref, w1_ref, degp_ref, g_ref, ns_ref, nd_ref):
    dp = degp_ref[...]
    deg_o = dp[:, 0:1] + dp[:, 1:2]
    deg_i = dp[:, 2:3] + dp[:, 3:4]
    ns = lax.rsqrt(jnp.maximum(deg_o, 1.0))
    nd = lax.rsqrt(jnp.maximum(deg_i, 1.0))
    ns_ref[...] = ns
    nd_ref[...] = nd
    g_ref[...] = jnp.dot(feat_ref[...], w1_ref[...],
                         preferred_element_type=jnp.float32) * ns


_tca = pl.pallas_call(
    _tca_body,
    grid=(_G,),
    in_specs=[
        pl.BlockSpec((_R, _DIN), lambda i: (i, 0)),
        pl.BlockSpec((_DIN, _H), lambda i: (0, 0)),
        pl.BlockSpec((_R, 4), lambda i: (i, 0)),
    ],
    out_specs=[
        pl.BlockSpec((_R, _H), lambda i: (i, 0)),
        pl.BlockSpec((_R, 1), lambda i: (i, 0)),
        pl.BlockSpec((_R, 1), lambda i: (i, 0)),
    ],
    out_shape=[
        jax.ShapeDtypeStruct((_N, _H), jnp.float32),
        jax.ShapeDtypeStruct((_N, 1), jnp.float32),
        jax.ShapeDtypeStruct((_N, 1), jnp.float32),
    ],
)


def _tcb_body_g(p_ref, nd_ref, ns_ref, b_ref, w_ref, g_ref):
    p = p_ref[...]
    h = jnp.maximum((p[0] + p[1]) * nd_ref[...] + b_ref[...], 0.0)
    g_ref[...] = jnp.dot(h, w_ref[...],
                         preferred_element_type=jnp.float32) * ns_ref[...]


def _tcb_body_hg(p_ref, nd_ref, ns_ref, b_ref, w_ref, h_ref, g_ref):
    p = p_ref[...]
    h = jnp.maximum((p[0] + p[1]) * nd_ref[...] + b_ref[...], 0.0)
    h_ref[...] = h
    g_ref[...] = jnp.dot(h, w_ref[...],
                         preferred_element_type=jnp.float32) * ns_ref[...]


_tcb_in_specs = [
    pl.BlockSpec((_NC, _R, _H), lambda i: (0, i, 0)),
    pl.BlockSpec((_R, 1), lambda i: (i, 0)),
    pl.BlockSpec((_R, 1), lambda i: (i, 0)),
    pl.BlockSpec((1, _H), lambda i: (0, 0)),
    pl.BlockSpec((_H, _H), lambda i: (0, 0)),
]

_tcb1 = pl.pallas_call(
    _tcb_body_g,
    grid=(_G,),
    in_specs=_tcb_in_specs,
    out_specs=pl.BlockSpec((_R, _H), lambda i: (i, 0)),
    out_shape=jax.ShapeDtypeStruct((_N, _H), jnp.float32),
)

_tcb2 = pl.pallas_call(
    _tcb_body_hg,
    grid=(_G,),
    in_specs=_tcb_in_specs,
    out_specs=[
        pl.BlockSpec((_R, _H), lambda i: (i, 0)),
        pl.BlockSpec((_R, _H), lambda i: (i, 0)),
    ],
    out_shape=[
        jax.ShapeDtypeStruct((_N, _H), jnp.float32),
        jax.ShapeDtypeStruct((_N, _H), jnp.float32),
    ],
)


def _tcc_body(p_ref, nd_ref, b_ref, h2_ref, wf_ref, bf_ref, o_ref):
    p = p_ref[...]
    h3 = jnp.maximum((p[0] + p[1]) * nd_ref[...] + b_ref[...], 0.0)
    y = jnp.maximum(h3 + h2_ref[...], 0.0)
    o_ref[...] = jnp.dot(y, wf_ref[...],
                         preferred_element_type=jnp.float32) + bf_ref[...]


_tcc = pl.pallas_call(
    _tcc_body,
    grid=(_G,),
    in_specs=[
        pl.BlockSpec((_NC, _R, _H), lambda i: (0, i, 0)),
        pl.BlockSpec((_R, 1), lambda i: (i, 0)),
        pl.BlockSpec((1, _H), lambda i: (0, 0)),
        pl.BlockSpec((_R, _H), lambda i: (i, 0)),
        pl.BlockSpec((_H, _C), lambda i: (0, 0)),
        pl.BlockSpec((1, _C), lambda i: (0, 0)),
    ],
    out_specs=pl.BlockSpec((_R, _C), lambda i: (i, 0)),
    out_shape=jax.ShapeDtypeStruct((_N, _C), jnp.float32),
)


def kernel(features, edge_index, W1, b1, W2, b2, W3, b3, Wf, bf):
    src = edge_index[0]
    dst = edge_index[1]
    src3 = src.reshape(_NW, _NCHUNK, _CH)
    dst3 = dst.reshape(_NW, _NCHUNK, _CH)

    degp = _deg(src3, dst3)
    degp_t = jnp.transpose(degp.reshape(2 * _NC, _N2))

    g1, ns, nd = _tca(features, W1, degp_t)

    p1 = _agg(g1, src3, dst3)
    g2 = _tcb1(p1, nd, ns, b1.reshape(1, _H), W2)
    p2 = _agg(g2, src3, dst3)
    h2, g3 = _tcb2(p2, nd, ns, b2.reshape(1, _H), W3)
    p3 = _agg(g3, src3, dst3)
    out = _tcc(p3, nd, b3.reshape(1, _H), h2, Wf, bf.reshape(1, _C))
    return out


# speedup vs baseline: 19.0866x; 1.1982x over previous
import jax
import jax.numpy as jnp
from jax import lax
from jax.experimental import pallas as pl
from jax.experimental.pallas import tpu as pltpu
from jax.experimental.pallas import tpu_sc as plsc

_N = 10000
_E = 640000
_DIN = 602
_H = 128
_C = 41

_NC = 2
_NS = 16
_NW = _NC * _NS
_N2 = 10240
_DSTRIPE = _N2 // _NS
_EC = _E // _NW
_CH = 100
_NCHUNK = _EC // _CH
_IB = 8
_NBLK = _NCHUNK // _IB
_NR = 10240
_RSTRIPE = _NR // _NS
_ZR = 16
_DLAG = 8
_R = 1000
_G = _N // _R

_mesh = plsc.VectorSubcoreMesh(core_axis_name="c", subcore_axis_name="s")


def _deg_body(src3_hbm, dst3_hbm, out_hbm, sidx2, didx2, ones_v, zbuf,
              acc_o, acc_i, sem_o, sem_i):
    cid = lax.axis_index("c")
    sid = lax.axis_index("s")
    wid = cid * _NS + sid

    pltpu.sync_copy(src3_hbm.at[wid], sidx2)
    pltpu.sync_copy(dst3_hbm.at[wid], didx2)

    for j in range(128 // 16):
        ones_v[pl.ds(j * 16, 16)] = jnp.ones((16,), jnp.float32)
    for j in range(_DSTRIPE // 16):
        zbuf[pl.ds(j * 16, 16)] = jnp.zeros((16,), jnp.float32)
    d0 = sid * _DSTRIPE
    pltpu.sync_copy(zbuf, acc_o.at[pl.ds(d0, _DSTRIPE)])
    pltpu.sync_copy(zbuf, acc_i.at[pl.ds(d0, _DSTRIPE)])
    plsc.subcore_barrier()

    ones_ch = ones_v.at[pl.ds(0, _CH)]

    def chunk(j, carry):
        pltpu.async_copy(ones_ch, acc_o.at[sidx2.at[j]], sem_o, add=True)
        pltpu.async_copy(ones_ch, acc_i.at[didx2.at[j]], sem_i, add=True)

        @pl.when(j >= _DLAG)
        def _():
            pltpu.make_async_copy(ones_ch, acc_o.at[sidx2.at[j]], sem_o).wait()
            pltpu.make_async_copy(ones_ch, acc_i.at[didx2.at[j]], sem_i).wait()

        return carry

    lax.fori_loop(0, _NCHUNK, chunk, 0)
    for _ in range(_DLAG):
        pltpu.make_async_copy(ones_ch, acc_o.at[sidx2.at[0]], sem_o).wait()
        pltpu.make_async_copy(ones_ch, acc_i.at[didx2.at[0]], sem_i).wait()
    plsc.subcore_barrier()

    pltpu.sync_copy(acc_o.at[pl.ds(d0, _DSTRIPE)],
                    out_hbm.at[0, cid, pl.ds(d0, _DSTRIPE)])
    pltpu.sync_copy(acc_i.at[pl.ds(d0, _DSTRIPE)],
                    out_hbm.at[1, cid, pl.ds(d0, _DSTRIPE)])


_deg = pl.kernel(
    _deg_body,
    out_type=jax.ShapeDtypeStruct((2, _NC, _N2), jnp.float32),
    mesh=_mesh,
    scratch_types=[
        pltpu.VMEM((_NCHUNK, _CH), jnp.int32),
        pltpu.VMEM((_NCHUNK, _CH), jnp.int32),
        pltpu.VMEM((128,), jnp.float32),
        pltpu.VMEM((_DSTRIPE,), jnp.float32),
        pltpu.VMEM_SHARED((_N2,), jnp.float32),
        pltpu.VMEM_SHARED((_N2,), jnp.float32),
        pltpu.SemaphoreType.DMA,
        pltpu.SemaphoreType.DMA,
    ],
)


def _agg_body(g_hbm, src3_hbm, dst3_hbm, out_hbm,
              s0, s1, d0i, d1i, b0, b1, b2, zrows, acc, sem_i, sem_g, sem_s):
    cid = lax.axis_index("c")
    sid = lax.axis_index("s")
    wid = cid * _NS + sid
    sblk = [s0, s1]
    dblk = [d0i, d1i]
    bufs = [b0, b1, b2]

    def zrow(i, carry):
        for j in range(_H // 16):
            zrows[i, pl.ds(j * 16, 16)] = jnp.zeros((16,), jnp.float32)
        return carry

    lax.fori_loop(0, _ZR, zrow, 0)
    r0 = sid * _RSTRIPE
    for k in range(_RSTRIPE // _ZR):
        pltpu.async_copy(zrows, acc.at[pl.ds(r0 + k * _ZR, _ZR)], sem_g)
    pltpu.async_copy(src3_hbm.at[wid, pl.ds(0, _IB)], s0, sem_i)
    pltpu.async_copy(dst3_hbm.at[wid, pl.ds(0, _IB)], d0i, sem_i)
    for k in range(_RSTRIPE // _ZR):
        pltpu.make_async_copy(zrows, acc.at[pl.ds(r0 + k * _ZR, _ZR)],
                              sem_g).wait()
    plsc.subcore_barrier()

    def block(kb, p):
        sb = sblk[p]
        db = dblk[p]
        j0 = kb * _IB
        pltpu.make_async_copy(src3_hbm.at[wid, pl.ds(j0, _IB)], sb,
                              sem_i).wait()
        pltpu.make_async_copy(dst3_hbm.at[wid, pl.ds(j0, _IB)], db,
                              sem_i).wait()

        @pl.when(kb + 1 < _NBLK)
        def _():
            pltpu.async_copy(src3_hbm.at[wid, pl.ds(j0 + _IB, _IB)],
                             sblk[1 - p], sem_i)
            pltpu.async_copy(dst3_hbm.at[wid, pl.ds(j0 + _IB, _IB)],
                             dblk[1 - p], sem_i)

        pltpu.async_copy(g_hbm.at[sb.at[0]], bufs[0], sem_g)

        for jj in range(_IB):
            bj = jj % 3
            j = j0 + jj
            pltpu.make_async_copy(g_hbm.at[sb.at[jj]], bufs[bj],
                                  sem_g).wait()
            pltpu.async_copy(bufs[bj], acc.at[db.at[jj]], sem_s, add=True)

            @pl.when(j >= 1)
            def _():
                pltpu.make_async_copy(bufs[(jj + 2) % 3],
                                      acc.at[db.at[jj]], sem_s).wait()

            if jj == 0:
                if _IB > 1:
                    pltpu.async_copy(g_hbm.at[sb.at[1]], bufs[1], sem_g)
                if _IB > 2:
                    pltpu.async_copy(g_hbm.at[sb.at[2]], bufs[2], sem_g)
            elif jj + 2 < _IB:
                pltpu.async_copy(g_hbm.at[sb.at[jj + 2]],
                                 bufs[(jj + 2) % 3], sem_g)

    def outer(jo, carry):
        block(jo * 2, 0)
        block(jo * 2 + 1, 1)
        return carry

    lax.fori_loop(0, _NBLK // 2, outer, 0)
    if _NBLK % 2:
        block(_NBLK - 1, 0)
    pltpu.make_async_copy(b0, acc.at[d0i.at[0]], sem_s).wait()
    plsc.subcore_barrier()

    pltpu.sync_copy(acc.at[pl.ds(r0, _RSTRIPE)],
                    out_hbm.at[cid, pl.ds(r0, _RSTRIPE)])


_agg = pl.kernel(
    _agg_body,
    out_type=jax.ShapeDtypeStruct((_NC, _NR, _H), jnp.float32),
    mesh=_mesh,
    scratch_types=[
        pltpu.VMEM((_IB, _CH), jnp.int32),
        pltpu.VMEM((_IB, _CH), jnp.int32),
        pltpu.VMEM((_IB, _CH), jnp.int32),
        pltpu.VMEM((_IB, _CH), jnp.int32),
        pltpu.VMEM((_CH, _H), jnp.float32),
        pltpu.VMEM((_CH, _H), jnp.float32),
        pltpu.VMEM((_CH, _H), jnp.float32),
        pltpu.VMEM((_ZR, _H), jnp.float32),
        pltpu.VMEM_SHARED((_NR, _H), jnp.float32),
        pltpu.SemaphoreType.DMA,
        pltpu.SemaphoreType.DMA,
        pltpu.SemaphoreType.DMA,
    ],
)


def _tca_body(feat_ref, w1_ref, degp_ref, g_ref, ns_ref, nd_ref):
    dp = degp_ref[...]
    deg_o = dp[:, 0:1] + dp[:, 1:2]
    deg_i = dp[:, 2:3] + dp[:, 3:4]
    ns = lax.rsqrt(jnp.maximum(deg_o, 1.0))
    nd = lax.rsqrt(jnp.maximum(deg_i, 1.0))
    ns_ref[...] = ns
    nd_ref[...] = nd
    g_ref[...] = jnp.dot(feat_ref[...], w1_ref[...],
                         preferred_element_type=jnp.float32) * ns


_tca = pl.pallas_call(
    _tca_body,
    grid=(_G,),
    in_specs=[
        pl.BlockSpec((_R, _DIN), lambda i: (i, 0)),
        pl.BlockSpec((_DIN, _H), lambda i: (0, 0)),
        pl.BlockSpec((_R, 4), lambda i: (i, 0)),
    ],
    out_specs=[
        pl.BlockSpec((_R, _H), lambda i: (i, 0)),
        pl.BlockSpec((_R, 1), lambda i: (i, 0)),
        pl.BlockSpec((_R, 1), lambda i: (i, 0)),
    ],
    out_shape=[
        jax.ShapeDtypeStruct((_N, _H), jnp.float32),
        jax.ShapeDtypeStruct((_N, 1), jnp.float32),
        jax.ShapeDtypeStruct((_N, 1), jnp.float32),
    ],
)


def _tcb_body_g(p_ref, nd_ref, ns_ref, b_ref, w_ref, g_ref):
    p = p_ref[...]
    h = jnp.maximum((p[0] + p[1]) * nd_ref[...] + b_ref[...], 0.0)
    g_ref[...] = jnp.dot(h, w_ref[...],
                         preferred_element_type=jnp.float32) * ns_ref[...]


def _tcb_body_hg(p_ref, nd_ref, ns_ref, b_ref, w_ref, h_ref, g_ref):
    p = p_ref[...]
    h = jnp.maximum((p[0] + p[1]) * nd_ref[...] + b_ref[...], 0.0)
    h_ref[...] = h
    g_ref[...] = jnp.dot(h, w_ref[...],
                         preferred_element_type=jnp.float32) * ns_ref[...]


_tcb_in_specs = [
    pl.BlockSpec((_NC, _R, _H), lambda i: (0, i, 0)),
    pl.BlockSpec((_R, 1), lambda i: (i, 0)),
    pl.BlockSpec((_R, 1), lambda i: (i, 0)),
    pl.BlockSpec((1, _H), lambda i: (0, 0)),
    pl.BlockSpec((_H, _H), lambda i: (0, 0)),
]

_tcb1 = pl.pallas_call(
    _tcb_body_g,
    grid=(_G,),
    in_specs=_tcb_in_specs,
    out_specs=pl.BlockSpec((_R, _H), lambda i: (i, 0)),
    out_shape=jax.ShapeDtypeStruct((_N, _H), jnp.float32),
)

_tcb2 = pl.pallas_call(
    _tcb_body_hg,
    grid=(_G,),
    in_specs=_tcb_in_specs,
    out_specs=[
        pl.BlockSpec((_R, _H), lambda i: (i, 0)),
        pl.BlockSpec((_R, _H), lambda i: (i, 0)),
    ],
    out_shape=[
        jax.ShapeDtypeStruct((_N, _H), jnp.float32),
        jax.ShapeDtypeStruct((_N, _H), jnp.float32),
    ],
)


def _tcc_body(p_ref, nd_ref, b_ref, h2_ref, wf_ref, bf_ref, o_ref):
    p = p_ref[...]
    h3 = jnp.maximum((p[0] + p[1]) * nd_ref[...] + b_ref[...], 0.0)
    y = jnp.maximum(h3 + h2_ref[...], 0.0)
    o_ref[...] = jnp.dot(y, wf_ref[...],
                         preferred_element_type=jnp.float32) + bf_ref[...]


_tcc = pl.pallas_call(
    _tcc_body,
    grid=(_G,),
    in_specs=[
        pl.BlockSpec((_NC, _R, _H), lambda i: (0, i, 0)),
        pl.BlockSpec((_R, 1), lambda i: (i, 0)),
        pl.BlockSpec((1, _H), lambda i: (0, 0)),
        pl.BlockSpec((_R, _H), lambda i: (i, 0)),
        pl.BlockSpec((_H, _C), lambda i: (0, 0)),
        pl.BlockSpec((1, _C), lambda i: (0, 0)),
    ],
    out_specs=pl.BlockSpec((_R, _C), lambda i: (i, 0)),
    out_shape=jax.ShapeDtypeStruct((_N, _C), jnp.float32),
)


def kernel(features, edge_index, W1, b1, W2, b2, W3, b3, Wf, bf):
    src = edge_index[0]
    dst = edge_index[1]
    src3 = src.reshape(_NW, _NCHUNK, _CH)
    dst3 = dst.reshape(_NW, _NCHUNK, _CH)

    degp = _deg(src3, dst3)
    degp_t = jnp.transpose(degp.reshape(2 * _NC, _N2))

    g1, ns, nd = _tca(features, W1, degp_t)

    p1 = _agg(g1, src3, dst3)
    g2 = _tcb1(p1, nd, ns, b1.reshape(1, _H), W2)
    p2 = _agg(g2, src3, dst3)
    h2, g3 = _tcb2(p2, nd, ns, b2.reshape(1, _H), W3)
    p3 = _agg(g3, src3, dst3)
    out = _tcc(p3, nd, b3.reshape(1, _H), h2, Wf, bf.reshape(1, _C))
    return out
